# Initial kernel scaffold; baseline (speedup 1.0000x reference)
import jax
import jax.numpy as jnp
from jax.experimental import pallas as pl


def _broadcast_body(w_ref, o_ref):
    o_ref[...] = jnp.broadcast_to(w_ref[None, ...], o_ref.shape)


def kernel(x, W):
    B, P, D = x.shape
    BB = 8
    out = pl.pallas_call(
        _broadcast_body,
        grid=(B // BB,),
        in_specs=[pl.BlockSpec((P, D), lambda i: (0, 0))],
        out_specs=pl.BlockSpec((BB, P, D), lambda i: (i, 0, 0)),
        out_shape=jax.ShapeDtypeStruct((B, P, D), W.dtype),
    )(W)
    return out


# speedup vs baseline: 4.2722x; 4.2722x over previous
import jax
import jax.numpy as jnp
from jax.experimental import pallas as pl


def _broadcast_body(w_ref, o_ref):
    o_ref[...] = jnp.broadcast_to(w_ref[...][None], o_ref.shape)


def kernel(x, W):
    B, P, D = x.shape
    BB = 8
    out = pl.pallas_call(
        _broadcast_body,
        grid=(B // BB,),
        in_specs=[pl.BlockSpec((P, D), lambda i: (0, 0))],
        out_specs=pl.BlockSpec((BB, P, D), lambda i: (i, 0, 0)),
        out_shape=jax.ShapeDtypeStruct((B, P, D), W.dtype),
    )(W)
    return out
